# baseline (speedup 1.0000x reference)
import functools

import jax
import jax.numpy as jnp
from jax.experimental import pallas as pl
from jax.experimental.pallas import tpu as pltpu


def _router_body(x_ref, w0_ref, b0_ref, w1_ref, b1_ref, w2_ref, b2_ref,
                 out_ref, m_ref, z_ref, basis_ref, *, s_t: int, tpb: int):
    s = pl.program_id(1)

    h = jnp.dot(x_ref[0], w0_ref[...], preferred_element_type=jnp.float32)
    h = jnp.maximum(h + b0_ref[...], 0.0)
    h = jnp.dot(h, w1_ref[...], preferred_element_type=jnp.float32)
    h = jnp.maximum(h + b1_ref[...], 0.0)
    logits = jnp.dot(h, w2_ref[...], preferred_element_type=jnp.float32)
    logits = logits + b2_ref[...]

    if tpb == 1:
        m_new = jnp.max(logits, axis=0, keepdims=True)
        e = jnp.exp(logits - m_new)
        out_ref[0] = e / jnp.sum(e, axis=0, keepdims=True)
        return

    m_old = jnp.where(s == 0, -jnp.inf, m_ref[...])
    z_old = jnp.where(s == 0, 0.0, z_ref[...])
    m_tile = jnp.max(logits, axis=0, keepdims=True)
    m_new = jnp.maximum(m_old, m_tile)
    e = jnp.exp(logits - m_new)
    z_new = z_old * jnp.exp(m_old - m_new) + jnp.sum(e, axis=0, keepdims=True)
    m_ref[...] = m_new
    z_ref[...] = z_new

    @pl.when(s < tpb - 1)
    def _store_raw():
        basis_ref[pl.ds(s, 1), :] = m_new
        out_ref[0, pl.ds(s * s_t, s_t), :] = e

    @pl.when(s == tpb - 1)
    def _finalize():
        inv_z = 1.0 / z_new
        out_ref[0, (tpb - 1) * s_t:, :] = e * inv_z
        for j in range(tpb - 1):
            seg = out_ref[0, j * s_t:(j + 1) * s_t, :]
            factor = jnp.exp(basis_ref[j:j + 1, :] - m_new) * inv_z
            out_ref[0, j * s_t:(j + 1) * s_t, :] = seg * factor


@jax.jit
def kernel(x, W0, b0, W1, b1, W2, b2):
    B, S, D = x.shape
    H0 = W0.shape[1]
    H1 = W1.shape[1]
    E = W2.shape[1]
    S_T = 2048
    tpb = S // S_T

    b0r = b0.reshape(1, H0)
    b1r = b1.reshape(1, H1)
    b2r = b2.reshape(1, E)

    body = functools.partial(_router_body, s_t=S_T, tpb=tpb)
    return pl.pallas_call(
        body,
        grid=(B, tpb),
        in_specs=[
            pl.BlockSpec((1, S_T, D), lambda b, s: (b, s, 0)),
            pl.BlockSpec((D, H0), lambda b, s: (0, 0)),
            pl.BlockSpec((1, H0), lambda b, s: (0, 0)),
            pl.BlockSpec((H0, H1), lambda b, s: (0, 0)),
            pl.BlockSpec((1, H1), lambda b, s: (0, 0)),
            pl.BlockSpec((H1, E), lambda b, s: (0, 0)),
            pl.BlockSpec((1, E), lambda b, s: (0, 0)),
        ],
        out_specs=pl.BlockSpec((1, S, E), lambda b, s: (b, 0, 0)),
        out_shape=jax.ShapeDtypeStruct((B, S, E), jnp.float32),
        scratch_shapes=[
            pltpu.VMEM((1, E), jnp.float32),
            pltpu.VMEM((1, E), jnp.float32),
            pltpu.VMEM((tpb, E), jnp.float32),
        ],
        compiler_params=pltpu.CompilerParams(
            dimension_semantics=("arbitrary", "arbitrary"),
            vmem_limit_bytes=100 * 1024 * 1024,
        ),
    )(x, W0, b0r, W1, b1r, W2, b2r)


# speedup vs baseline: 1.0070x; 1.0070x over previous
import functools

import jax
import jax.numpy as jnp
from jax.experimental import pallas as pl
from jax.experimental.pallas import tpu as pltpu


def _router_body(x_ref, w0_ref, b0_ref, w1_ref, b1_ref, w2_ref, b2_ref,
                 out_ref, m_ref, z_ref, basis_ref, *, s_t: int, tpb: int):
    s = pl.program_id(1)

    h = jnp.dot(x_ref[0], w0_ref[...], preferred_element_type=jnp.float32)
    h = jnp.maximum(h + b0_ref[...], 0.0)
    h = jnp.dot(h, w1_ref[...], preferred_element_type=jnp.float32)
    h = jnp.maximum(h + b1_ref[...], 0.0)
    logits = jnp.dot(h, w2_ref[...], preferred_element_type=jnp.float32)
    logits = logits + b2_ref[...]

    if tpb == 1:
        m_new = jnp.max(logits, axis=0, keepdims=True)
        e = jnp.exp(logits - m_new)
        out_ref[0] = e / jnp.sum(e, axis=0, keepdims=True)
        return

    m_old = jnp.where(s == 0, -jnp.inf, m_ref[...])
    z_old = jnp.where(s == 0, 0.0, z_ref[...])
    m_tile = jnp.max(logits, axis=0, keepdims=True)
    m_new = jnp.maximum(m_old, m_tile)
    e = jnp.exp(logits - m_new)
    z_new = z_old * jnp.exp(m_old - m_new) + jnp.sum(e, axis=0, keepdims=True)
    m_ref[...] = m_new
    z_ref[...] = z_new

    @pl.when(s < tpb - 1)
    def _store_raw():
        basis_ref[pl.ds(s, 1), :] = m_new
        out_ref[0, pl.ds(s * s_t, s_t), :] = e

    @pl.when(s == tpb - 1)
    def _finalize():
        inv_z = 1.0 / z_new
        out_ref[0, (tpb - 1) * s_t:, :] = e * inv_z
        for j in range(tpb - 1):
            seg = out_ref[0, j * s_t:(j + 1) * s_t, :]
            factor = jnp.exp(basis_ref[j:j + 1, :] - m_new) * inv_z
            out_ref[0, j * s_t:(j + 1) * s_t, :] = seg * factor


@jax.jit
def kernel(x, W0, b0, W1, b1, W2, b2):
    B, S, D = x.shape
    H0 = W0.shape[1]
    H1 = W1.shape[1]
    E = W2.shape[1]
    S_T = 1024
    tpb = S // S_T

    b0r = b0.reshape(1, H0)
    b1r = b1.reshape(1, H1)
    b2r = b2.reshape(1, E)

    body = functools.partial(_router_body, s_t=S_T, tpb=tpb)
    return pl.pallas_call(
        body,
        grid=(B, tpb),
        in_specs=[
            pl.BlockSpec((1, S_T, D), lambda b, s: (b, s, 0)),
            pl.BlockSpec((D, H0), lambda b, s: (0, 0)),
            pl.BlockSpec((1, H0), lambda b, s: (0, 0)),
            pl.BlockSpec((H0, H1), lambda b, s: (0, 0)),
            pl.BlockSpec((1, H1), lambda b, s: (0, 0)),
            pl.BlockSpec((H1, E), lambda b, s: (0, 0)),
            pl.BlockSpec((1, E), lambda b, s: (0, 0)),
        ],
        out_specs=pl.BlockSpec((1, S, E), lambda b, s: (b, 0, 0)),
        out_shape=jax.ShapeDtypeStruct((B, S, E), jnp.float32),
        scratch_shapes=[
            pltpu.VMEM((1, E), jnp.float32),
            pltpu.VMEM((1, E), jnp.float32),
            pltpu.VMEM((tpb, E), jnp.float32),
        ],
        compiler_params=pltpu.CompilerParams(
            dimension_semantics=("arbitrary", "arbitrary"),
            vmem_limit_bytes=100 * 1024 * 1024,
        ),
    )(x, W0, b0r, W1, b1r, W2, b2r)
